# Initial kernel scaffold; baseline (speedup 1.0000x reference)
import jax
import jax.numpy as jnp
from jax.experimental import pallas as pl


def kernel(xyz, features, params):
    raise NotImplementedError("write your pallas kernel here")


# speedup vs baseline: 12.0468x; 12.0468x over previous
import functools

import jax
import jax.numpy as jnp
from jax import lax
from jax.experimental import pallas as pl
from jax.experimental.pallas import tpu as pltpu
from jax.experimental.pallas import tpu_sc as plsc

D = 64
K = 16
PB = 256
PP = 128

_HI = jax.lax.Precision.HIGHEST


def _dot(a, b, precision=_HI):
    return jax.lax.dot_general(
        a, b, (((1,), (0,)), ((), ())),
        precision=precision, preferred_element_type=jnp.float32)


def _knn_body(xyzp_ref, xyzT_ref, feat_ref, fc1w_ref, fc1b_ref, wq_ref,
              idx_ref, x_ref, q_ref):
    b = pl.program_id(0)
    n = xyzT_ref.shape[2]
    xi = xyzp_ref[0]
    xT = xyzT_ref[0]
    cross = _dot(xi, xT, precision=jax.lax.Precision.DEFAULT)
    rsum_j = jnp.sum(xT * xT, axis=0, keepdims=True)
    rsum_i = jnp.sum(xi * xi, axis=1, keepdims=True)
    d = rsum_i + rsum_j - 2.0 * cross
    iota = jax.lax.broadcasted_iota(jnp.int32, (PB, n), 1)
    cols = []
    for _ in range(K):
        m = jnp.min(d, axis=1, keepdims=True)
        idx = jnp.min(jnp.where(d == m, iota, n), axis=1, keepdims=True)
        cols.append(idx)
        d = jnp.where(iota == idx, jnp.float32(jnp.inf), d)
    idx_ref[0] = jnp.concatenate(cols, axis=1) + b * n
    x = _dot(feat_ref[0], fc1w_ref[...]) + fc1b_ref[...]
    x_ref[0] = jnp.concatenate(
        [x, xi, jnp.zeros((PB, 56), jnp.float32)], axis=1)
    q_ref[0] = _dot(x, wq_ref[...])


def _knn_call(xyzp, xyzT, feat, fc1w, fc1b, wq):
    B, N, _ = xyzp.shape
    grid = (B, N // PB)
    return pl.pallas_call(
        _knn_body,
        grid=grid,
        in_specs=[
            pl.BlockSpec((1, PB, 8), lambda b, i: (b, i, 0)),
            pl.BlockSpec((1, 8, N), lambda b, i: (b, 0, 0)),
            pl.BlockSpec((1, PB, D), lambda b, i: (b, i, 0)),
            pl.BlockSpec((D, D), lambda b, i: (0, 0)),
            pl.BlockSpec((1, D), lambda b, i: (0, 0)),
            pl.BlockSpec((D, D), lambda b, i: (0, 0)),
        ],
        out_specs=[
            pl.BlockSpec((1, PB, K), lambda b, i: (b, i, 0)),
            pl.BlockSpec((1, PB, 128), lambda b, i: (b, i, 0)),
            pl.BlockSpec((1, PB, D), lambda b, i: (b, i, 0)),
        ],
        out_shape=[
            jax.ShapeDtypeStruct((B, N, K), jnp.int32),
            jax.ShapeDtypeStruct((B, N, 128), jnp.float32),
            jax.ShapeDtypeStruct((B, N, D), jnp.float32),
        ],
    )(xyzp, xyzT, feat, fc1w, fc1b, wq)


_GCHUNK = 128


def _sc_gather(xt, idx_flat):
    M = idx_flat.shape[0]
    W = xt.shape[1]
    info = plsc.get_sparse_core_info()
    NW = info.num_cores * info.num_subcores
    per_w = M // NW
    steps = per_w // _GCHUNK
    mesh = plsc.VectorSubcoreMesh(core_axis_name="c", subcore_axis_name="s")

    @functools.partial(
        pl.kernel,
        mesh=mesh,
        out_type=jax.ShapeDtypeStruct((M, W), jnp.float32),
        scratch_types=[
            pltpu.VMEM((_GCHUNK,), jnp.int32),
            pltpu.VMEM((_GCHUNK, W), jnp.float32),
            pltpu.SemaphoreType.DMA,
        ],
    )
    def gather_k(xt_hbm, idx_hbm, gx_out, idx_v, rx, sem1):
        wid = lax.axis_index("s") * info.num_cores + lax.axis_index("c")
        base = wid * per_w

        def step(i, carry):
            off = base + i * _GCHUNK
            pltpu.sync_copy(idx_hbm.at[pl.ds(off, _GCHUNK)], idx_v)
            pltpu.async_copy(xt_hbm.at[idx_v], rx, sem1).wait()
            pltpu.sync_copy(rx, gx_out.at[pl.ds(off, _GCHUNK)])
            return carry

        lax.fori_loop(0, steps, step, 0)

    return gather_k(xt, idx_flat)


def _rep16(a, pp):
    c = a.shape[-1]
    return jnp.broadcast_to(a[:, None, :], (pp, K, c)).reshape(pp * K, c)


def _dense_body(q_ref, gx_ref, cxyz_ref, feat_ref,
                wk_ref, wv_ref, d1w_ref, d1b_ref, d2w_ref, d2b_ref,
                g1w_ref, g1b_ref, g2w_ref, g2b_ref, fc2w_ref, fc2b_ref,
                attn_ref, res_ref):
    xj = gx_ref[:, :D]
    gz = gx_ref[:, D:D + 8]
    kj = _dot(xj, wk_ref[...])
    vj = _dot(xj, wv_ref[...])
    rel = _rep16(cxyz_ref[...], PP) - gz
    pos = jnp.maximum(_dot(rel, d1w_ref[...]) + d1b_ref[...], 0.0)
    pos = _dot(pos, d2w_ref[...]) + d2b_ref[...]
    g = _rep16(q_ref[...], PP) - kj + pos
    a = jnp.maximum(_dot(g, g1w_ref[...]) + g1b_ref[...], 0.0)
    a = _dot(a, g2w_ref[...]) + g2b_ref[...]
    a3 = a.reshape(PP, K, D) * jnp.float32(0.125)
    m = jnp.max(a3, axis=1, keepdims=True)
    e = jnp.exp(a3 - m)
    s = jnp.sum(e, axis=1, keepdims=True)
    p3 = e / s
    attn_ref[...] = p3.reshape(PP * K, D)
    w = p3 * (vj + pos).reshape(PP, K, D)
    r = jnp.sum(w, axis=1)
    res_ref[...] = _dot(r, fc2w_ref[...]) + fc2b_ref[...] + feat_ref[...]


def _dense_call(q2, gx, xyz2, feat2, p):
    BN = q2.shape[0]
    grid = (BN // PP,)
    wfull = lambda shape: pl.BlockSpec(shape, lambda i: (0, 0))
    return pl.pallas_call(
        _dense_body,
        grid=grid,
        in_specs=[
            pl.BlockSpec((PP, D), lambda i: (i, 0)),
            pl.BlockSpec((PP * K, 128), lambda i: (i, 0)),
            pl.BlockSpec((PP, 8), lambda i: (i, 0)),
            pl.BlockSpec((PP, D), lambda i: (i, 0)),
            wfull((D, D)), wfull((D, D)),
            wfull((8, D)), wfull((1, D)), wfull((D, D)), wfull((1, D)),
            wfull((D, D)), wfull((1, D)), wfull((D, D)), wfull((1, D)),
            wfull((D, D)), wfull((1, D)),
        ],
        out_specs=[
            pl.BlockSpec((PP * K, D), lambda i: (i, 0)),
            pl.BlockSpec((PP, D), lambda i: (i, 0)),
        ],
        out_shape=[
            jax.ShapeDtypeStruct((BN * K, D), jnp.float32),
            jax.ShapeDtypeStruct((BN, D), jnp.float32),
        ],
    )(q2, gx, xyz2, feat2,
      p['wk'], p['wv'],
      p['delta1_w'], p['delta1_b'], p['delta2_w'], p['delta2_b'],
      p['gamma1_w'], p['gamma1_b'], p['gamma2_w'], p['gamma2_b'],
      p['fc2_w'], p['fc2_b'])


def kernel(xyz, features, params):
    p = params
    B, N, _ = xyz.shape
    BN = B * N
    xyzp = jnp.pad(xyz, ((0, 0), (0, 0), (0, 5)))
    xyzT = jnp.swapaxes(xyzp, 1, 2)
    fc1b = p['fc1_b'].reshape(1, D)

    idx3, xt3, q3 = _knn_call(xyzp, xyzT, features, p['fc1_w'], fc1b, p['wq'])

    idx_flat = idx3.reshape(BN * K)
    xt = xt3.reshape(BN, 128)
    q2 = q3.reshape(BN, D)
    xyz2 = xyzp.reshape(BN, 8)

    gx = _sc_gather(xt, idx_flat)

    d1w = jnp.pad(p['delta1_w'], ((0, 5), (0, 0)))
    pr = {
        'wk': p['wk'], 'wv': p['wv'],
        'delta1_w': d1w, 'delta1_b': p['delta1_b'].reshape(1, D),
        'delta2_w': p['delta2_w'], 'delta2_b': p['delta2_b'].reshape(1, D),
        'gamma1_w': p['gamma1_w'], 'gamma1_b': p['gamma1_b'].reshape(1, D),
        'gamma2_w': p['gamma2_w'], 'gamma2_b': p['gamma2_b'].reshape(1, D),
        'fc2_w': p['fc2_w'], 'fc2_b': p['fc2_b'].reshape(1, D),
    }
    attn2, res2 = _dense_call(q2, gx, xyz2, features.reshape(BN, D), pr)

    return res2.reshape(B, N, D), attn2.reshape(B, N, K, D)


# speedup vs baseline: 17.5764x; 1.4590x over previous
import functools

import jax
import jax.numpy as jnp
from jax import lax
from jax.experimental import pallas as pl
from jax.experimental.pallas import tpu as pltpu
from jax.experimental.pallas import tpu_sc as plsc

D = 64
K = 16
PB = 256
PP = 128

_HI = jax.lax.Precision.DEFAULT


def _dot(a, b, precision=_HI):
    return jax.lax.dot_general(
        a, b, (((1,), (0,)), ((), ())),
        precision=precision, preferred_element_type=jnp.float32)


def _knn_body(xyzp_ref, xyzT_ref, feat_ref, fc1w_ref, fc1b_ref, wq_ref,
              idx_ref, x_ref, q_ref):
    b = pl.program_id(0)
    n = xyzT_ref.shape[2]
    xi = xyzp_ref[0]
    xT = xyzT_ref[0]
    cross = _dot(xi, xT, precision=jax.lax.Precision.DEFAULT)
    rsum_j = jnp.sum(xT * xT, axis=0, keepdims=True)
    rsum_i = jnp.sum(xi * xi, axis=1, keepdims=True)
    d = rsum_i + rsum_j - 2.0 * cross
    iota = jax.lax.broadcasted_iota(jnp.int32, (PB, n), 1)
    cols = []
    for _ in range(K):
        m = jnp.min(d, axis=1, keepdims=True)
        idx = jnp.min(jnp.where(d == m, iota, n), axis=1, keepdims=True)
        cols.append(idx)
        d = jnp.where(iota == idx, jnp.float32(jnp.inf), d)
    idx_ref[0] = jnp.concatenate(cols, axis=1) + b * n
    x = _dot(feat_ref[0], fc1w_ref[...]) + fc1b_ref[...]
    x_ref[0] = jnp.concatenate(
        [x, xi, jnp.zeros((PB, 56), jnp.float32)], axis=1)
    q_ref[0] = _dot(x, wq_ref[...])


def _knn_call(xyzp, xyzT, feat, fc1w, fc1b, wq):
    B, N, _ = xyzp.shape
    grid = (B, N // PB)
    return pl.pallas_call(
        _knn_body,
        grid=grid,
        in_specs=[
            pl.BlockSpec((1, PB, 8), lambda b, i: (b, i, 0)),
            pl.BlockSpec((1, 8, N), lambda b, i: (b, 0, 0)),
            pl.BlockSpec((1, PB, D), lambda b, i: (b, i, 0)),
            pl.BlockSpec((D, D), lambda b, i: (0, 0)),
            pl.BlockSpec((1, D), lambda b, i: (0, 0)),
            pl.BlockSpec((D, D), lambda b, i: (0, 0)),
        ],
        out_specs=[
            pl.BlockSpec((1, PB, K), lambda b, i: (b, i, 0)),
            pl.BlockSpec((1, PB, 128), lambda b, i: (b, i, 0)),
            pl.BlockSpec((1, PB, D), lambda b, i: (b, i, 0)),
        ],
        out_shape=[
            jax.ShapeDtypeStruct((B, N, K), jnp.int32),
            jax.ShapeDtypeStruct((B, N, 128), jnp.float32),
            jax.ShapeDtypeStruct((B, N, D), jnp.float32),
        ],
    )(xyzp, xyzT, feat, fc1w, fc1b, wq)


_GCHUNK = 128


def _sc_gather(xt, idx_flat):
    M = idx_flat.shape[0]
    W = xt.shape[1]
    info = plsc.get_sparse_core_info()
    NW = info.num_cores * info.num_subcores
    per_w = M // NW
    steps = per_w // _GCHUNK
    mesh = plsc.VectorSubcoreMesh(core_axis_name="c", subcore_axis_name="s")

    @functools.partial(
        pl.kernel,
        mesh=mesh,
        out_type=jax.ShapeDtypeStruct((M, W), jnp.float32),
        scratch_types=[
            pltpu.VMEM((_GCHUNK,), jnp.int32),
            pltpu.VMEM((_GCHUNK, W), jnp.float32),
            pltpu.SemaphoreType.DMA,
        ],
    )
    def gather_k(xt_hbm, idx_hbm, gx_out, idx_v, rx, sem1):
        wid = lax.axis_index("s") * info.num_cores + lax.axis_index("c")
        base = wid * per_w

        def step(i, carry):
            off = base + i * _GCHUNK
            pltpu.sync_copy(idx_hbm.at[pl.ds(off, _GCHUNK)], idx_v)
            pltpu.async_copy(xt_hbm.at[idx_v], rx, sem1).wait()
            pltpu.sync_copy(rx, gx_out.at[pl.ds(off, _GCHUNK)])
            return carry

        lax.fori_loop(0, steps, step, 0)

    return gather_k(xt, idx_flat)


def _rep16(a, pp):
    c = a.shape[-1]
    return jnp.broadcast_to(a[:, None, :], (pp, K, c)).reshape(pp * K, c)


def _dense_body(q_ref, gx_ref, cxyz_ref, feat_ref,
                wk_ref, wv_ref, d1w_ref, d1b_ref, d2w_ref, d2b_ref,
                g1w_ref, g1b_ref, g2w_ref, g2b_ref, fc2w_ref, fc2b_ref,
                attn_ref, res_ref):
    xj = gx_ref[:, :D]
    gz = gx_ref[:, D:D + 8]
    kj = _dot(xj, wk_ref[...])
    vj = _dot(xj, wv_ref[...])
    rel = _rep16(cxyz_ref[...], PP) - gz
    pos = jnp.maximum(_dot(rel, d1w_ref[...]) + d1b_ref[...], 0.0)
    pos = _dot(pos, d2w_ref[...]) + d2b_ref[...]
    g = _rep16(q_ref[...], PP) - kj + pos
    a = jnp.maximum(_dot(g, g1w_ref[...]) + g1b_ref[...], 0.0)
    a = _dot(a, g2w_ref[...]) + g2b_ref[...]
    a3 = a.reshape(PP, K, D) * jnp.float32(0.125)
    m = jnp.max(a3, axis=1, keepdims=True)
    e = jnp.exp(a3 - m)
    s = jnp.sum(e, axis=1, keepdims=True)
    p3 = e / s
    attn_ref[...] = p3.reshape(PP * K, D)
    w = p3 * (vj + pos).reshape(PP, K, D)
    r = jnp.sum(w, axis=1)
    res_ref[...] = _dot(r, fc2w_ref[...]) + fc2b_ref[...] + feat_ref[...]


def _dense_call(q2, gx, xyz2, feat2, p):
    BN = q2.shape[0]
    grid = (BN // PP,)
    wfull = lambda shape: pl.BlockSpec(shape, lambda i: (0, 0))
    return pl.pallas_call(
        _dense_body,
        grid=grid,
        in_specs=[
            pl.BlockSpec((PP, D), lambda i: (i, 0)),
            pl.BlockSpec((PP * K, 128), lambda i: (i, 0)),
            pl.BlockSpec((PP, 8), lambda i: (i, 0)),
            pl.BlockSpec((PP, D), lambda i: (i, 0)),
            wfull((D, D)), wfull((D, D)),
            wfull((8, D)), wfull((1, D)), wfull((D, D)), wfull((1, D)),
            wfull((D, D)), wfull((1, D)), wfull((D, D)), wfull((1, D)),
            wfull((D, D)), wfull((1, D)),
        ],
        out_specs=[
            pl.BlockSpec((PP * K, D), lambda i: (i, 0)),
            pl.BlockSpec((PP, D), lambda i: (i, 0)),
        ],
        out_shape=[
            jax.ShapeDtypeStruct((BN * K, D), jnp.float32),
            jax.ShapeDtypeStruct((BN, D), jnp.float32),
        ],
    )(q2, gx, xyz2, feat2,
      p['wk'], p['wv'],
      p['delta1_w'], p['delta1_b'], p['delta2_w'], p['delta2_b'],
      p['gamma1_w'], p['gamma1_b'], p['gamma2_w'], p['gamma2_b'],
      p['fc2_w'], p['fc2_b'])


def kernel(xyz, features, params):
    p = params
    B, N, _ = xyz.shape
    BN = B * N
    xyzp = jnp.pad(xyz, ((0, 0), (0, 0), (0, 5)))
    xyzT = jnp.swapaxes(xyzp, 1, 2)
    fc1b = p['fc1_b'].reshape(1, D)

    idx3, xt3, q3 = _knn_call(xyzp, xyzT, features, p['fc1_w'], fc1b, p['wq'])

    idx_flat = idx3.reshape(BN * K)
    xt = xt3.reshape(BN, 128)
    q2 = q3.reshape(BN, D)
    xyz2 = xyzp.reshape(BN, 8)

    gx = _sc_gather(xt, idx_flat)

    d1w = jnp.pad(p['delta1_w'], ((0, 5), (0, 0)))
    pr = {
        'wk': p['wk'], 'wv': p['wv'],
        'delta1_w': d1w, 'delta1_b': p['delta1_b'].reshape(1, D),
        'delta2_w': p['delta2_w'], 'delta2_b': p['delta2_b'].reshape(1, D),
        'gamma1_w': p['gamma1_w'], 'gamma1_b': p['gamma1_b'].reshape(1, D),
        'gamma2_w': p['gamma2_w'], 'gamma2_b': p['gamma2_b'].reshape(1, D),
        'fc2_w': p['fc2_w'], 'fc2_b': p['fc2_b'].reshape(1, D),
    }
    attn2, res2 = _dense_call(q2, gx, xyz2, features.reshape(BN, D), pr)

    return res2.reshape(B, N, D), attn2.reshape(B, N, K, D)


# speedup vs baseline: 19.5541x; 1.1125x over previous
import functools

import jax
import jax.numpy as jnp
from jax import lax
from jax.experimental import pallas as pl
from jax.experimental.pallas import tpu as pltpu
from jax.experimental.pallas import tpu_sc as plsc

D = 64
K = 16
PB = 256
PP = 128

_HI = jax.lax.Precision.DEFAULT


def _dot(a, b, precision=_HI):
    return jax.lax.dot_general(
        a, b, (((1,), (0,)), ((), ())),
        precision=precision, preferred_element_type=jnp.float32)


def _knn_body(xyzp_ref, xyzT_ref, feat_ref, fc1w_ref, fc1b_ref, wq_ref,
              idx_ref, x_ref, q_ref):
    b = pl.program_id(0)
    n = xyzT_ref.shape[2]
    xi = xyzp_ref[0]
    xT = xyzT_ref[0]
    cross = _dot(xi, xT, precision=jax.lax.Precision.DEFAULT)
    rsum_j = jnp.sum(xT * xT, axis=0, keepdims=True)
    rsum_i = jnp.sum(xi * xi, axis=1, keepdims=True)
    d = rsum_i + rsum_j - 2.0 * cross
    iota = jax.lax.broadcasted_iota(jnp.int32, (PB, n), 1)
    cols = []
    for _ in range(K):
        idx = jnp.argmin(d, axis=1)[:, None]
        cols.append(idx)
        d = jnp.where(iota == idx, jnp.float32(jnp.inf), d)
    idx_ref[0] = jnp.concatenate(cols, axis=1) + b * n
    x = _dot(feat_ref[0], fc1w_ref[...]) + fc1b_ref[...]
    x_ref[0] = jnp.concatenate(
        [x, xi, jnp.zeros((PB, 56), jnp.float32)], axis=1)
    q_ref[0] = _dot(x, wq_ref[...])


def _knn_call(xyzp, xyzT, feat, fc1w, fc1b, wq):
    B, N, _ = xyzp.shape
    grid = (B, N // PB)
    return pl.pallas_call(
        _knn_body,
        grid=grid,
        in_specs=[
            pl.BlockSpec((1, PB, 8), lambda b, i: (b, i, 0)),
            pl.BlockSpec((1, 8, N), lambda b, i: (b, 0, 0)),
            pl.BlockSpec((1, PB, D), lambda b, i: (b, i, 0)),
            pl.BlockSpec((D, D), lambda b, i: (0, 0)),
            pl.BlockSpec((1, D), lambda b, i: (0, 0)),
            pl.BlockSpec((D, D), lambda b, i: (0, 0)),
        ],
        out_specs=[
            pl.BlockSpec((1, PB, K), lambda b, i: (b, i, 0)),
            pl.BlockSpec((1, PB, 128), lambda b, i: (b, i, 0)),
            pl.BlockSpec((1, PB, D), lambda b, i: (b, i, 0)),
        ],
        out_shape=[
            jax.ShapeDtypeStruct((B, N, K), jnp.int32),
            jax.ShapeDtypeStruct((B, N, 128), jnp.float32),
            jax.ShapeDtypeStruct((B, N, D), jnp.float32),
        ],
    )(xyzp, xyzT, feat, fc1w, fc1b, wq)


_GCHUNK = 128


def _sc_gather(xt, idx_flat):
    M = idx_flat.shape[0]
    W = xt.shape[1]
    info = plsc.get_sparse_core_info()
    NW = info.num_cores * info.num_subcores
    per_w = M // NW
    steps = per_w // _GCHUNK
    mesh = plsc.VectorSubcoreMesh(core_axis_name="c", subcore_axis_name="s")

    @functools.partial(
        pl.kernel,
        mesh=mesh,
        out_type=jax.ShapeDtypeStruct((M, W), jnp.float32),
        scratch_types=[
            pltpu.VMEM((_GCHUNK,), jnp.int32),
            pltpu.VMEM((_GCHUNK, W), jnp.float32),
            pltpu.SemaphoreType.DMA,
        ],
    )
    def gather_k(xt_hbm, idx_hbm, gx_out, idx_v, rx, sem1):
        wid = lax.axis_index("s") * info.num_cores + lax.axis_index("c")
        base = wid * per_w

        def step(i, carry):
            off = base + i * _GCHUNK
            pltpu.sync_copy(idx_hbm.at[pl.ds(off, _GCHUNK)], idx_v)
            pltpu.async_copy(xt_hbm.at[idx_v], rx, sem1).wait()
            pltpu.sync_copy(rx, gx_out.at[pl.ds(off, _GCHUNK)])
            return carry

        lax.fori_loop(0, steps, step, 0)

    return gather_k(xt, idx_flat)


def _rep16(a, pp):
    c = a.shape[-1]
    return jnp.broadcast_to(a[:, None, :], (pp, K, c)).reshape(pp * K, c)


def _dense_body(q_ref, gx_ref, cxyz_ref, feat_ref,
                wk_ref, wv_ref, d1w_ref, d1b_ref, d2w_ref, d2b_ref,
                g1w_ref, g1b_ref, g2w_ref, g2b_ref, fc2w_ref, fc2b_ref,
                attn_ref, res_ref):
    xj = gx_ref[:, :D]
    gz = gx_ref[:, D:D + 8]
    kj = _dot(xj, wk_ref[...])
    vj = _dot(xj, wv_ref[...])
    rel = _rep16(cxyz_ref[...], PP) - gz
    pos = jnp.maximum(_dot(rel, d1w_ref[...]) + d1b_ref[...], 0.0)
    pos = _dot(pos, d2w_ref[...]) + d2b_ref[...]
    g = _rep16(q_ref[...], PP) - kj + pos
    a = jnp.maximum(_dot(g, g1w_ref[...]) + g1b_ref[...], 0.0)
    a = _dot(a, g2w_ref[...]) + g2b_ref[...]
    a3 = a.reshape(PP, K, D) * jnp.float32(0.125)
    m = jnp.max(a3, axis=1, keepdims=True)
    e = jnp.exp(a3 - m)
    s = jnp.sum(e, axis=1, keepdims=True)
    p3 = e / s
    attn_ref[...] = p3.reshape(PP * K, D)
    w = p3 * (vj + pos).reshape(PP, K, D)
    r = jnp.sum(w, axis=1)
    res_ref[...] = _dot(r, fc2w_ref[...]) + fc2b_ref[...] + feat_ref[...]


def _dense_call(q2, gx, xyz2, feat2, p):
    BN = q2.shape[0]
    grid = (BN // PP,)
    wfull = lambda shape: pl.BlockSpec(shape, lambda i: (0, 0))
    return pl.pallas_call(
        _dense_body,
        grid=grid,
        in_specs=[
            pl.BlockSpec((PP, D), lambda i: (i, 0)),
            pl.BlockSpec((PP * K, 128), lambda i: (i, 0)),
            pl.BlockSpec((PP, 8), lambda i: (i, 0)),
            pl.BlockSpec((PP, D), lambda i: (i, 0)),
            wfull((D, D)), wfull((D, D)),
            wfull((8, D)), wfull((1, D)), wfull((D, D)), wfull((1, D)),
            wfull((D, D)), wfull((1, D)), wfull((D, D)), wfull((1, D)),
            wfull((D, D)), wfull((1, D)),
        ],
        out_specs=[
            pl.BlockSpec((PP * K, D), lambda i: (i, 0)),
            pl.BlockSpec((PP, D), lambda i: (i, 0)),
        ],
        out_shape=[
            jax.ShapeDtypeStruct((BN * K, D), jnp.float32),
            jax.ShapeDtypeStruct((BN, D), jnp.float32),
        ],
    )(q2, gx, xyz2, feat2,
      p['wk'], p['wv'],
      p['delta1_w'], p['delta1_b'], p['delta2_w'], p['delta2_b'],
      p['gamma1_w'], p['gamma1_b'], p['gamma2_w'], p['gamma2_b'],
      p['fc2_w'], p['fc2_b'])


def kernel(xyz, features, params):
    p = params
    B, N, _ = xyz.shape
    BN = B * N
    xyzp = jnp.pad(xyz, ((0, 0), (0, 0), (0, 5)))
    xyzT = jnp.swapaxes(xyzp, 1, 2)
    fc1b = p['fc1_b'].reshape(1, D)

    idx3, xt3, q3 = _knn_call(xyzp, xyzT, features, p['fc1_w'], fc1b, p['wq'])

    idx_flat = idx3.reshape(BN * K)
    xt = xt3.reshape(BN, 128)
    q2 = q3.reshape(BN, D)
    xyz2 = xyzp.reshape(BN, 8)

    gx = _sc_gather(xt, idx_flat)

    d1w = jnp.pad(p['delta1_w'], ((0, 5), (0, 0)))
    pr = {
        'wk': p['wk'], 'wv': p['wv'],
        'delta1_w': d1w, 'delta1_b': p['delta1_b'].reshape(1, D),
        'delta2_w': p['delta2_w'], 'delta2_b': p['delta2_b'].reshape(1, D),
        'gamma1_w': p['gamma1_w'], 'gamma1_b': p['gamma1_b'].reshape(1, D),
        'gamma2_w': p['gamma2_w'], 'gamma2_b': p['gamma2_b'].reshape(1, D),
        'fc2_w': p['fc2_w'], 'fc2_b': p['fc2_b'].reshape(1, D),
    }
    attn2, res2 = _dense_call(q2, gx, xyz2, features.reshape(BN, D), pr)

    return res2.reshape(B, N, D), attn2.reshape(B, N, K, D)


# speedup vs baseline: 19.9896x; 1.0223x over previous
import functools

import jax
import jax.numpy as jnp
from jax import lax
from jax.experimental import pallas as pl
from jax.experimental.pallas import tpu as pltpu
from jax.experimental.pallas import tpu_sc as plsc

D = 64
K = 16
PB = 256
PP = 128

_DEF = jax.lax.Precision.DEFAULT


def _dot(a, b):
    return jax.lax.dot_general(
        a, b, (((1,), (0,)), ((), ())),
        precision=_DEF, preferred_element_type=jnp.float32)


def _knn_body(xyzp_ref, xyzT_ref, feat_ref, fc1w_ref, fc1b_ref, wq_ref,
              idx_ref, x_ref, q_ref):
    n = xyzT_ref.shape[1]
    xi = xyzp_ref[...]
    xT = xyzT_ref[...]
    cross = _dot(xi, xT)
    rsum_j = jnp.sum(xT * xT, axis=0, keepdims=True)
    rsum_i = jnp.sum(xi * xi, axis=1, keepdims=True)
    d = rsum_i + rsum_j - 2.0 * cross
    iota = jax.lax.broadcasted_iota(jnp.int32, (PB, n), 1)
    cols = []
    for _ in range(K):
        idx = jnp.argmin(d, axis=1)[:, None]
        cols.append(idx)
        d = jnp.where(iota == idx, jnp.float32(jnp.inf), d)
    idx_ref[...] = jnp.concatenate(cols, axis=1)
    x = _dot(feat_ref[...], fc1w_ref[...]) + fc1b_ref[...]
    x_ref[...] = jnp.concatenate(
        [x, xi, jnp.zeros((PB, 56), jnp.float32)], axis=1)
    q_ref[...] = _dot(x, wq_ref[...])


def _knn_call(xyzp, xyzT, feat, fc1w, fc1b, wq):
    N = xyzp.shape[0]
    grid = (N // PB,)
    return pl.pallas_call(
        _knn_body,
        grid=grid,
        in_specs=[
            pl.BlockSpec((PB, 8), lambda i: (i, 0)),
            pl.BlockSpec((8, N), lambda i: (0, 0)),
            pl.BlockSpec((PB, D), lambda i: (i, 0)),
            pl.BlockSpec((D, D), lambda i: (0, 0)),
            pl.BlockSpec((1, D), lambda i: (0, 0)),
            pl.BlockSpec((D, D), lambda i: (0, 0)),
        ],
        out_specs=[
            pl.BlockSpec((PB, K), lambda i: (i, 0)),
            pl.BlockSpec((PB, 128), lambda i: (i, 0)),
            pl.BlockSpec((PB, D), lambda i: (i, 0)),
        ],
        out_shape=[
            jax.ShapeDtypeStruct((N, K), jnp.int32),
            jax.ShapeDtypeStruct((N, 128), jnp.float32),
            jax.ShapeDtypeStruct((N, D), jnp.float32),
        ],
    )(xyzp, xyzT, feat, fc1w, fc1b, wq)


def _sc_gather(xt, idx2d):
    R = idx2d.shape[0]
    W = xt.shape[1]
    M = R * 128
    info = plsc.get_sparse_core_info()
    NW = info.num_cores * info.num_subcores
    rows_per_w = R // NW
    mesh = plsc.VectorSubcoreMesh(core_axis_name="c", subcore_axis_name="s")

    @functools.partial(
        pl.kernel,
        mesh=mesh,
        out_type=jax.ShapeDtypeStruct((M, W), jnp.float32),
        scratch_types=[
            pltpu.VMEM((128,), jnp.int32),
            pltpu.VMEM((128, W), jnp.float32),
            pltpu.SemaphoreType.DMA,
        ],
    )
    def gather_k(xt_hbm, idx_hbm, gx_out, idx_v, rx, sem1):
        wid = lax.axis_index("s") * info.num_cores + lax.axis_index("c")
        base = wid * rows_per_w

        def step(i, carry):
            r = base + i
            pltpu.sync_copy(idx_hbm.at[r], idx_v)
            pltpu.async_copy(xt_hbm.at[idx_v], rx, sem1).wait()
            pltpu.sync_copy(rx, gx_out.at[pl.ds(r * 128, 128)])
            return carry

        lax.fori_loop(0, rows_per_w, step, 0)

    return gather_k(xt, idx2d)


def _rep16(a, pp):
    c = a.shape[-1]
    return jnp.broadcast_to(a[:, None, :], (pp, K, c)).reshape(pp * K, c)


def _dense_body(q_ref, gx_ref, cxyz_ref, feat_ref,
                wk_ref, wv_ref, d1w_ref, d1b_ref, d2w_ref, d2b_ref,
                g1w_ref, g1b_ref, g2w_ref, g2b_ref, fc2w_ref, fc2b_ref,
                attn_ref, res_ref):
    xj = gx_ref[:, :D]
    gz = gx_ref[:, D:D + 8]
    kj = _dot(xj, wk_ref[...])
    vj = _dot(xj, wv_ref[...])
    rel = _rep16(cxyz_ref[...], PP) - gz
    pos = jnp.maximum(_dot(rel, d1w_ref[...]) + d1b_ref[...], 0.0)
    pos = _dot(pos, d2w_ref[...]) + d2b_ref[...]
    g = _rep16(q_ref[...], PP) - kj + pos
    a = jnp.maximum(_dot(g, g1w_ref[...]) + g1b_ref[...], 0.0)
    a = _dot(a, g2w_ref[...]) + g2b_ref[...]
    a3 = a.reshape(PP, K, D) * jnp.float32(0.125)
    m = jnp.max(a3, axis=1, keepdims=True)
    e = jnp.exp(a3 - m)
    s = jnp.sum(e, axis=1, keepdims=True)
    p3 = e / s
    attn_ref[...] = p3
    w = p3 * (vj + pos).reshape(PP, K, D)
    r = jnp.sum(w, axis=1)
    res_ref[...] = _dot(r, fc2w_ref[...]) + fc2b_ref[...] + feat_ref[...]


def _dense_call(q2, gx, xyz2, feat2, p):
    N = q2.shape[0]
    grid = (N // PP,)
    wfull = lambda shape: pl.BlockSpec(shape, lambda i: (0, 0))
    return pl.pallas_call(
        _dense_body,
        grid=grid,
        in_specs=[
            pl.BlockSpec((PP, D), lambda i: (i, 0)),
            pl.BlockSpec((PP * K, 128), lambda i: (i, 0)),
            pl.BlockSpec((PP, 8), lambda i: (i, 0)),
            pl.BlockSpec((PP, D), lambda i: (i, 0)),
            wfull((D, D)), wfull((D, D)),
            wfull((8, D)), wfull((1, D)), wfull((D, D)), wfull((1, D)),
            wfull((D, D)), wfull((1, D)), wfull((D, D)), wfull((1, D)),
            wfull((D, D)), wfull((1, D)),
        ],
        out_specs=[
            pl.BlockSpec((PP, K, D), lambda i: (i, 0, 0)),
            pl.BlockSpec((PP, D), lambda i: (i, 0)),
        ],
        out_shape=[
            jax.ShapeDtypeStruct((N, K, D), jnp.float32),
            jax.ShapeDtypeStruct((N, D), jnp.float32),
        ],
    )(q2, gx, xyz2, feat2,
      p['wk'], p['wv'],
      p['delta1_w'], p['delta1_b'], p['delta2_w'], p['delta2_b'],
      p['gamma1_w'], p['gamma1_b'], p['gamma2_w'], p['gamma2_b'],
      p['fc2_w'], p['fc2_b'])


def kernel(xyz, features, params):
    p = params
    B, N, _ = xyz.shape
    xyzp = jnp.pad(xyz, ((0, 0), (0, 0), (0, 5)))
    xyzT = jnp.swapaxes(xyzp, 1, 2)
    fc1b = p['fc1_b'].reshape(1, D)
    d1w = jnp.pad(p['delta1_w'], ((0, 5), (0, 0)))
    pr = {
        'wk': p['wk'], 'wv': p['wv'],
        'delta1_w': d1w, 'delta1_b': p['delta1_b'].reshape(1, D),
        'delta2_w': p['delta2_w'], 'delta2_b': p['delta2_b'].reshape(1, D),
        'gamma1_w': p['gamma1_w'], 'gamma1_b': p['gamma1_b'].reshape(1, D),
        'gamma2_w': p['gamma2_w'], 'gamma2_b': p['gamma2_b'].reshape(1, D),
        'fc2_w': p['fc2_w'], 'fc2_b': p['fc2_b'].reshape(1, D),
    }

    knn = [_knn_call(xyzp[b], xyzT[b], features[b], p['fc1_w'], fc1b, p['wq'])
           for b in range(B)]
    gx = [_sc_gather(knn[b][1], knn[b][0].reshape(N * K // 128, 128))
          for b in range(B)]
    outs = [_dense_call(knn[b][2], gx[b], xyzp[b].reshape(N, 8),
                        features[b], pr) for b in range(B)]

    res = jnp.stack([o[1] for o in outs])
    attn = jnp.stack([o[0] for o in outs])
    return res, attn
